# baseline (speedup 1.0000x reference)
import functools

import jax
import jax.numpy as jnp
from jax import lax
from jax.experimental import pallas as pl
from jax.experimental.pallas import tpu as pltpu
from jax.experimental.pallas import tpu_sc as plsc

EMB = 128
B = 4096
L = 50

NC = 2
NS = 16
NW = NC * NS

ROWS = B * L
RPW = ROWS // NW
BPW = B // NW
BPC = 4
CHUNK = BPC * L
NCH = RPW // CHUNK
NBUF = 4

_mesh = plsc.VectorSubcoreMesh(core_axis_name="c", subcore_axis_name="s")


@functools.partial(
    pl.kernel,
    mesh=_mesh,
    out_type=jax.ShapeDtypeStruct((B, L, EMB), jnp.float32),
    compiler_params=pltpu.CompilerParams(use_tc_tiling_on_sc=True),
    scratch_types=[
        pltpu.VMEM((BPW, 128), jnp.int32),
    ]
    + [pltpu.VMEM((CHUNK + 1, EMB), jnp.float32) for _ in range(NBUF)]
    + [pltpu.SemaphoreType.DMA for _ in range(2 * NBUF)],
)
def _shifted_gather(x_hbm, table_hbm, out_hbm, x_v, *bufs_and_sems):
    bufs = bufs_and_sems[:NBUF]
    gsem = bufs_and_sems[NBUF : 2 * NBUF]
    osem = bufs_and_sems[2 * NBUF :]
    wid = lax.axis_index("s") * NC + lax.axis_index("c")
    pltpu.sync_copy(x_hbm.at[pl.ds(wid * BPW, BPW)], x_v)
    zeros16 = jnp.zeros((16,), jnp.float32)

    def group(g, carry):
        for s in range(NBUF):
            j = g * NBUF + s
            gb0 = wid * BPW + j * BPC

            @pl.when(g > 0)
            def _():
                for k in range(BPC):
                    pltpu.make_async_copy(
                        bufs[s].at[pl.ds(1 + k * L, L)], out_hbm.at[gb0 + k], osem[s]
                    ).wait()

            for k in range(BPC):
                pltpu.async_copy(
                    table_hbm.at[x_v.at[j * BPC + k, pl.ds(0, L)]],
                    bufs[s].at[pl.ds(k * L, L)],
                    gsem[s],
                )

        for s in range(NBUF):
            j = g * NBUF + s
            gb0 = wid * BPW + j * BPC
            for k in range(BPC):
                pltpu.make_async_copy(
                    table_hbm.at[x_v.at[j * BPC + k, pl.ds(0, L)]],
                    bufs[s].at[pl.ds(k * L, L)],
                    gsem[s],
                ).wait()
            for r in range(L, CHUNK + 1, L):
                for k in range(EMB // 16):
                    bufs[s][r, pl.ds(k * 16, 16)] = zeros16
            for k in range(BPC):
                pltpu.async_copy(
                    bufs[s].at[pl.ds(1 + k * L, L)], out_hbm.at[gb0 + k], osem[s]
                )
        return carry

    lax.fori_loop(0, NCH // NBUF, group, 0)
    for s in range(NBUF):
        gb0 = wid * BPW + (NCH - NBUF + s) * BPC
        for k in range(BPC):
            pltpu.make_async_copy(
                bufs[s].at[pl.ds(1 + k * L, L)], out_hbm.at[gb0 + k], osem[s]
            ).wait()


def kernel(x, table):
    xp = jnp.pad(x.astype(jnp.int32), ((0, 0), (0, 128 - L)))
    return _shifted_gather(xp, table)


# speedup vs baseline: 1.7780x; 1.7780x over previous
import functools

import jax
import jax.numpy as jnp
from jax import lax
from jax.experimental import pallas as pl
from jax.experimental.pallas import tpu as pltpu
from jax.experimental.pallas import tpu_sc as plsc

EMB = 128
B = 4096
L = 50

NC = 2
NS = 16
NW = NC * NS

W = B // NW
NBUF = 5

_mesh = plsc.VectorSubcoreMesh(core_axis_name="c", subcore_axis_name="s")


@functools.partial(
    pl.kernel,
    mesh=_mesh,
    out_type=jax.ShapeDtypeStruct((L, B, EMB), jnp.float32),
    scratch_types=[
        pltpu.VMEM((L, W), jnp.int32),
        pltpu.VMEM((W, EMB), jnp.float32),
    ]
    + [pltpu.VMEM((W, EMB), jnp.float32) for _ in range(NBUF)]
    + [pltpu.SemaphoreType.DMA for _ in range(2 * NBUF)],
)
def _shifted_gather(xt_hbm, table_hbm, out_hbm, x_v, zbuf, *bufs_and_sems):
    bufs = bufs_and_sems[:NBUF]
    gsem = bufs_and_sems[NBUF : 2 * NBUF]
    osem = bufs_and_sems[2 * NBUF :]
    wid = lax.axis_index("s") * NC + lax.axis_index("c")
    col0 = wid * W
    pltpu.sync_copy(xt_hbm.at[:, pl.ds(col0, W)], x_v)

    zeros16 = jnp.zeros((16,), jnp.float32)

    def zrow(r, carry):
        for k in range(EMB // 16):
            zbuf[r, pl.ds(k * 16, 16)] = zeros16
        return carry

    lax.fori_loop(0, W, zrow, 0)

    def group(g, carry):
        for s in range(NBUF):
            c = g * NBUF + s

            @pl.when(g > 0)
            def _():
                pltpu.make_async_copy(
                    bufs[s], out_hbm.at[c, pl.ds(col0, W)], osem[s]
                ).wait()

            @pl.when(c < L - 1)
            def _():
                pltpu.async_copy(table_hbm.at[x_v.at[c + 1]], bufs[s], gsem[s])

        for s in range(NBUF):
            c = g * NBUF + s

            @pl.when(c < L - 1)
            def _():
                pltpu.make_async_copy(
                    table_hbm.at[x_v.at[c + 1]], bufs[s], gsem[s]
                ).wait()
                pltpu.async_copy(bufs[s], out_hbm.at[c, pl.ds(col0, W)], osem[s])

            @pl.when(c == L - 1)
            def _():
                pltpu.async_copy(zbuf, out_hbm.at[c, pl.ds(col0, W)], osem[s])
        return carry

    lax.fori_loop(0, L // NBUF, group, 0)
    for s in range(NBUF):
        c = L - NBUF + s
        src = zbuf if s == NBUF - 1 else bufs[s]
        pltpu.make_async_copy(src, out_hbm.at[c, pl.ds(col0, W)], osem[s]).wait()


def kernel(x, table):
    xt = x.astype(jnp.int32).T
    return _shifted_gather(xt, table).transpose(1, 0, 2)


# speedup vs baseline: 1.7862x; 1.0046x over previous
import functools

import jax
import jax.numpy as jnp
from jax import lax
from jax.experimental import pallas as pl
from jax.experimental.pallas import tpu as pltpu
from jax.experimental.pallas import tpu_sc as plsc

EMB = 128
B = 4096
L = 50

NC = 2
NS = 16
NW = NC * NS

W = B // NW
NBUF = 5

_mesh = plsc.VectorSubcoreMesh(core_axis_name="c", subcore_axis_name="s")


@functools.partial(
    pl.kernel,
    mesh=_mesh,
    out_type=jax.ShapeDtypeStruct((L, B, EMB), jnp.float32),
    scratch_types=[
        pltpu.VMEM((L, W), jnp.int32),
        pltpu.VMEM((W, EMB), jnp.float32),
    ]
    + [pltpu.VMEM((W, EMB), jnp.float32) for _ in range(NBUF)]
    + [pltpu.SemaphoreType.DMA for _ in range(2 * NBUF)],
)
def _shifted_gather(xt_hbm, table_hbm, out_hbm, x_v, zbuf, *bufs_and_sems):
    bufs = bufs_and_sems[:NBUF]
    gsem = bufs_and_sems[NBUF : 2 * NBUF]
    osem = bufs_and_sems[2 * NBUF :]
    wid = lax.axis_index("s") * NC + lax.axis_index("c")
    col0 = wid * W
    idx_cp = pltpu.async_copy(xt_hbm.at[:, pl.ds(col0, W)], x_v, gsem[0])

    zeros16 = jnp.zeros((16,), jnp.float32)

    def zrow(r, carry):
        for k in range(EMB // 16):
            zbuf[r, pl.ds(k * 16, 16)] = zeros16
        return carry

    lax.fori_loop(0, W, zrow, 0)
    idx_cp.wait()

    def group(g, carry):
        for s in range(NBUF):
            c = g * NBUF + s

            @pl.when(g > 0)
            def _():
                pltpu.make_async_copy(
                    bufs[s], out_hbm.at[c, pl.ds(col0, W)], osem[s]
                ).wait()

            @pl.when(c < L - 1)
            def _():
                pltpu.async_copy(table_hbm.at[x_v.at[c + 1]], bufs[s], gsem[s])

        for s in range(NBUF):
            c = g * NBUF + s

            @pl.when(c < L - 1)
            def _():
                pltpu.make_async_copy(
                    table_hbm.at[x_v.at[c + 1]], bufs[s], gsem[s]
                ).wait()
                pltpu.async_copy(bufs[s], out_hbm.at[c, pl.ds(col0, W)], osem[s])

            @pl.when(c == L - 1)
            def _():
                pltpu.async_copy(zbuf, out_hbm.at[c, pl.ds(col0, W)], osem[s])
        return carry

    lax.fori_loop(0, L // NBUF, group, 0)
    for s in range(NBUF):
        c = L - NBUF + s
        src = zbuf if s == NBUF - 1 else bufs[s]
        pltpu.make_async_copy(src, out_hbm.at[c, pl.ds(col0, W)], osem[s]).wait()


def kernel(x, table):
    xt = x.astype(jnp.int32).T
    return _shifted_gather(xt, table).transpose(1, 0, 2)
